# baseline (speedup 1.0000x reference)
import functools

import jax
import jax.numpy as jnp
from jax import lax
from jax.experimental import pallas as pl
from jax.experimental.pallas import tpu as pltpu
from jax.experimental.pallas import tpu_sc as plsc

_N_TASKS = 1000
_N_SPLITS = 8
_N_SKILLS = 64
_TOPK = 3
_BATCH = 16384


def _table_body(ml_ref, out_ref):
    n = ml_ref.shape[0]
    iota = lax.broadcasted_iota(jnp.int32, (n, _N_SKILLS), 1)
    vals = []
    sels = []
    for s_grp in range(_TOPK):
        x = ml_ref[:, s_grp * _N_SKILLS:(s_grp + 1) * _N_SKILLS]
        m = jnp.max(x, axis=1, keepdims=True)
        e = jnp.exp(x - m)
        p = e / jnp.sum(e, axis=1, keepdims=True)
        w = p
        v_k = []
        s_k = []
        for _ in range(_TOPK):
            v = jnp.max(w, axis=1, keepdims=True)
            cand = jnp.where(w == v, iota, _N_SKILLS)
            i = jnp.min(cand, axis=1, keepdims=True)
            sel = iota == i
            v_k.append(v)
            s_k.append(sel)
            w = jnp.where(sel, -jnp.inf, w)
        vals.append(v_k)
        sels.append(s_k)
    outs = []
    for s in range(_TOPK):
        acc = jnp.zeros((n, _N_SKILLS), jnp.float32)
        for k in range(_TOPK):
            denom = vals[0][k] + vals[1][k] + vals[2][k]
            acc = acc + jnp.where(sels[s][k], vals[s][k] / denom, 0.0)
        outs.append(acc)
    outs.append(jnp.zeros((n, _N_SKILLS), jnp.float32))
    out_ref[...] = jnp.concatenate(outs, axis=1).astype(jnp.bfloat16)


def _build_table(module_logits):
    tab = pl.pallas_call(
        _table_body,
        out_shape=jax.ShapeDtypeStruct((_N_TASKS, 4 * _N_SKILLS), jnp.bfloat16),
    )(module_logits)
    return lax.bitcast_convert_type(
        tab.reshape(_N_TASKS, 2 * _N_SKILLS, 2), jnp.float32
    )


_D = 2 * _N_SKILLS
_NW = 32
_CHUNK = 128


def _make_gather_body(batch):
    b_per_w = batch // _NW
    n_chunks = max(1, b_per_w // _CHUNK)
    chunk = b_per_w // n_chunks

    def body(table_hbm, idx_hbm, out_hbm, idx_v, buf0, buf1, g0, g1, w0, w1):
        wid = lax.axis_index("s") * 2 + lax.axis_index("c")
        base = wid * b_per_w
        pltpu.sync_copy(idx_hbm.at[pl.ds(base, b_per_w)], idx_v)
        bufs = (buf0, buf1)
        gsems = (g0, g1)
        wsems = (w0, w1)

        def start_gather(c):
            return pltpu.async_copy(
                table_hbm.at[idx_v.at[pl.ds(c * chunk, chunk)]],
                bufs[c % 2],
                gsems[c % 2],
            )

        def start_write(c):
            return pltpu.async_copy(
                bufs[c % 2],
                out_hbm.at[pl.ds(base + c * chunk, chunk)],
                wsems[c % 2],
            )

        gathers = [None] * n_chunks
        writes = [None] * n_chunks
        gathers[0] = start_gather(0)
        for c in range(n_chunks):
            if c + 1 < n_chunks:
                if c >= 1:
                    writes[c - 1].wait()
                gathers[c + 1] = start_gather(c + 1)
            gathers[c].wait()
            writes[c] = start_write(c)
        for w in writes[-2:]:
            w.wait()

    return body, chunk, b_per_w


def _gather(table, task_ids):
    batch = task_ids.shape[0]
    body, chunk, b_per_w = _make_gather_body(batch)
    mesh = plsc.VectorSubcoreMesh(core_axis_name="c", subcore_axis_name="s")
    grab = pl.kernel(
        body,
        out_type=jax.ShapeDtypeStruct((batch, _D), jnp.float32),
        mesh=mesh,
        scratch_types=[
            pltpu.VMEM((b_per_w,), jnp.int32),
            pltpu.VMEM((chunk, _D), jnp.float32),
            pltpu.VMEM((chunk, _D), jnp.float32),
            pltpu.SemaphoreType.DMA,
            pltpu.SemaphoreType.DMA,
            pltpu.SemaphoreType.DMA,
            pltpu.SemaphoreType.DMA,
        ],
    )
    return grab(table, task_ids)


def _expand(flat):
    batch = flat.shape[0]
    bf = lax.bitcast_convert_type(flat, jnp.bfloat16)
    top = bf.reshape(batch, 4, _N_SKILLS).astype(jnp.float32)
    tail = jnp.zeros((batch, _N_SPLITS - 4, _N_SKILLS), jnp.float32)
    return jnp.concatenate([top, tail], axis=1)


def kernel(task_ids, module_logits):
    table = _build_table(module_logits)
    half = _BATCH // 2
    flat0 = _gather(table, task_ids[:half])
    flat1 = _gather(table, task_ids[half:])
    return jnp.concatenate([_expand(flat0), _expand(flat1)], axis=0)


# speedup vs baseline: 1.3326x; 1.3326x over previous
import functools

import jax
import jax.numpy as jnp
from jax import lax
from jax.experimental import pallas as pl
from jax.experimental.pallas import tpu as pltpu
from jax.experimental.pallas import tpu_sc as plsc

_N_TASKS = 1000
_N_SPLITS = 8
_N_SKILLS = 64
_TOPK = 3
_BATCH = 16384


def _table_body(ml_ref, out_ref):
    n = ml_ref.shape[0]
    iota = lax.broadcasted_iota(jnp.int32, (n, _N_SKILLS), 1)
    vals = []
    sels = []
    for s_grp in range(_TOPK):
        x = ml_ref[:, s_grp * _N_SKILLS:(s_grp + 1) * _N_SKILLS]
        m = jnp.max(x, axis=1, keepdims=True)
        e = jnp.exp(x - m)
        p = e / jnp.sum(e, axis=1, keepdims=True)
        w = p
        v_k = []
        s_k = []
        for _ in range(_TOPK):
            v = jnp.max(w, axis=1, keepdims=True)
            cand = jnp.where(w == v, iota, _N_SKILLS)
            i = jnp.min(cand, axis=1, keepdims=True)
            sel = iota == i
            v_k.append(v)
            s_k.append(sel)
            w = jnp.where(sel, -jnp.inf, w)
        vals.append(v_k)
        sels.append(s_k)
    outs = []
    for s in range(_TOPK):
        acc = jnp.zeros((n, _N_SKILLS), jnp.float32)
        for k in range(_TOPK):
            denom = vals[0][k] + vals[1][k] + vals[2][k]
            acc = acc + jnp.where(sels[s][k], vals[s][k] / denom, 0.0)
        outs.append(acc)
    outs.append(jnp.zeros((n, _N_SKILLS), jnp.float32))
    out_ref[...] = jnp.concatenate(outs, axis=1)


def _build_table(module_logits):
    return pl.pallas_call(
        _table_body,
        out_shape=jax.ShapeDtypeStruct((_N_TASKS, 4 * _N_SKILLS), jnp.float32),
    )(module_logits)


_D = 4 * _N_SKILLS
_NW = 32
_CHUNK = 128


def _make_gather_body(batch):
    b_per_w = batch // _NW
    n_chunks = max(1, b_per_w // _CHUNK)
    chunk = b_per_w // n_chunks

    def body(table_hbm, idx_hbm, out_hbm, idx_v, buf0, buf1, g0, g1, w0, w1):
        wid = lax.axis_index("s") * 2 + lax.axis_index("c")
        base = wid * b_per_w
        pltpu.sync_copy(idx_hbm.at[pl.ds(base, b_per_w)], idx_v)
        bufs = (buf0, buf1)
        gsems = (g0, g1)
        wsems = (w0, w1)

        def start_gather(c):
            return pltpu.async_copy(
                table_hbm.at[idx_v.at[pl.ds(c * chunk, chunk)]],
                bufs[c % 2],
                gsems[c % 2],
            )

        def start_write(c):
            return pltpu.async_copy(
                bufs[c % 2],
                out_hbm.at[pl.ds(base + c * chunk, chunk)],
                wsems[c % 2],
            )

        gathers = [None] * n_chunks
        writes = [None] * n_chunks
        gathers[0] = start_gather(0)
        for c in range(n_chunks):
            if c + 1 < n_chunks:
                if c >= 1:
                    writes[c - 1].wait()
                gathers[c + 1] = start_gather(c + 1)
            gathers[c].wait()
            writes[c] = start_write(c)
        for w in writes[-2:]:
            w.wait()

    return body, chunk, b_per_w


def _gather(table, task_ids):
    batch = task_ids.shape[0]
    body, chunk, b_per_w = _make_gather_body(batch)
    mesh = plsc.VectorSubcoreMesh(core_axis_name="c", subcore_axis_name="s")
    grab = pl.kernel(
        body,
        out_type=jax.ShapeDtypeStruct((batch, _D), jnp.float32),
        mesh=mesh,
        scratch_types=[
            pltpu.VMEM((b_per_w,), jnp.int32),
            pltpu.VMEM((chunk, _D), jnp.float32),
            pltpu.VMEM((chunk, _D), jnp.float32),
            pltpu.SemaphoreType.DMA,
            pltpu.SemaphoreType.DMA,
            pltpu.SemaphoreType.DMA,
            pltpu.SemaphoreType.DMA,
        ],
    )
    return grab(table, task_ids)


def _expand(flat):
    batch = flat.shape[0]
    top = flat.reshape(batch, 4, _N_SKILLS)
    tail = jnp.zeros((batch, _N_SPLITS - 4, _N_SKILLS), jnp.float32)
    return jnp.concatenate([top, tail], axis=1)


def kernel(task_ids, module_logits):
    table = _build_table(module_logits)
    n_pieces = 4
    piece = _BATCH // n_pieces
    outs = []
    for p in range(n_pieces):
        flat = _gather(table, task_ids[p * piece:(p + 1) * piece])
        outs.append(_expand(flat))
    return jnp.concatenate(outs, axis=0)


# speedup vs baseline: 1.3840x; 1.0386x over previous
import functools

import jax
import jax.numpy as jnp
from jax import lax
from jax.experimental import pallas as pl
from jax.experimental.pallas import tpu as pltpu
from jax.experimental.pallas import tpu_sc as plsc

_N_TASKS = 1000
_N_SPLITS = 8
_N_SKILLS = 64
_TOPK = 3
_BATCH = 16384


def _table_body(ml_ref, out_ref):
    n = ml_ref.shape[0]
    iota = lax.broadcasted_iota(jnp.int32, (n, _N_SKILLS), 1)
    vals = []
    sels = []
    for s_grp in range(_TOPK):
        x = ml_ref[:, s_grp * _N_SKILLS:(s_grp + 1) * _N_SKILLS]
        m = jnp.max(x, axis=1, keepdims=True)
        e = jnp.exp(x - m)
        p = e / jnp.sum(e, axis=1, keepdims=True)
        w = p
        v_k = []
        s_k = []
        for _ in range(_TOPK):
            v = jnp.max(w, axis=1, keepdims=True)
            cand = jnp.where(w == v, iota, _N_SKILLS)
            i = jnp.min(cand, axis=1, keepdims=True)
            sel = iota == i
            v_k.append(v)
            s_k.append(sel)
            w = jnp.where(sel, -jnp.inf, w)
        vals.append(v_k)
        sels.append(s_k)
    outs = []
    for s in range(_TOPK):
        acc = jnp.zeros((n, _N_SKILLS), jnp.float32)
        for k in range(_TOPK):
            denom = vals[0][k] + vals[1][k] + vals[2][k]
            acc = acc + jnp.where(sels[s][k], vals[s][k] / denom, 0.0)
        outs.append(acc)
    outs.append(jnp.zeros((n, _N_SKILLS), jnp.float32))
    out_ref[...] = jnp.concatenate(outs, axis=1)


def _build_table(module_logits):
    return pl.pallas_call(
        _table_body,
        out_shape=jax.ShapeDtypeStruct((_N_TASKS, 4 * _N_SKILLS), jnp.float32),
    )(module_logits)


_D = 4 * _N_SKILLS
_NW = 32
_CHUNK = 128


def _make_gather_body(batch):
    b_per_w = batch // _NW
    n_chunks = max(1, b_per_w // _CHUNK)
    chunk = b_per_w // n_chunks

    def body(table_hbm, idx_hbm, out_hbm, idx_v, buf0, buf1, g0, g1, w0, w1):
        wid = lax.axis_index("s") * 2 + lax.axis_index("c")
        base = wid * b_per_w
        pltpu.sync_copy(idx_hbm.at[pl.ds(base, b_per_w)], idx_v)
        bufs = (buf0, buf1)
        gsems = (g0, g1)
        wsems = (w0, w1)

        def start_gather(c):
            return pltpu.async_copy(
                table_hbm.at[idx_v.at[pl.ds(c * chunk, chunk)]],
                bufs[c % 2],
                gsems[c % 2],
            )

        def start_write(c):
            return pltpu.async_copy(
                bufs[c % 2],
                out_hbm.at[pl.ds(base + c * chunk, chunk)],
                wsems[c % 2],
            )

        gathers = [None] * n_chunks
        writes = [None] * n_chunks
        gathers[0] = start_gather(0)
        for c in range(n_chunks):
            if c + 1 < n_chunks:
                if c >= 1:
                    writes[c - 1].wait()
                gathers[c + 1] = start_gather(c + 1)
            gathers[c].wait()
            writes[c] = start_write(c)
        for w in writes[-2:]:
            w.wait()

    return body, chunk, b_per_w


def _gather(table, task_ids):
    batch = task_ids.shape[0]
    body, chunk, b_per_w = _make_gather_body(batch)
    mesh = plsc.VectorSubcoreMesh(core_axis_name="c", subcore_axis_name="s")
    grab = pl.kernel(
        body,
        out_type=jax.ShapeDtypeStruct((batch, _D), jnp.float32),
        mesh=mesh,
        scratch_types=[
            pltpu.VMEM((b_per_w,), jnp.int32),
            pltpu.VMEM((chunk, _D), jnp.float32),
            pltpu.VMEM((chunk, _D), jnp.float32),
            pltpu.SemaphoreType.DMA,
            pltpu.SemaphoreType.DMA,
            pltpu.SemaphoreType.DMA,
            pltpu.SemaphoreType.DMA,
        ],
    )
    return grab(table, task_ids)


def _expand(flat):
    batch = flat.shape[0]
    top = flat.reshape(batch, 4, _N_SKILLS)
    tail = jnp.zeros((batch, _N_SPLITS - 4, _N_SKILLS), jnp.float32)
    return jnp.concatenate([top, tail], axis=1)


def kernel(task_ids, module_logits):
    table = _build_table(module_logits)
    n_pieces = 2
    piece = _BATCH // n_pieces
    outs = []
    for p in range(n_pieces):
        flat = _gather(table, task_ids[p * piece:(p + 1) * piece])
        outs.append(_expand(flat))
    return jnp.concatenate(outs, axis=0)


# speedup vs baseline: 1.3901x; 1.0044x over previous
import jax
import jax.numpy as jnp
from jax import lax
from jax.experimental import pallas as pl
from jax.experimental.pallas import tpu as pltpu
from jax.experimental.pallas import tpu_sc as plsc

_N_TASKS = 1000
_N_SPLITS = 8
_N_SKILLS = 64
_TOPK = 3
_BATCH = 16384


def _table_body(ml_ref, out_ref):
    n = ml_ref.shape[0]
    iota = lax.broadcasted_iota(jnp.int32, (n, _N_SKILLS), 1)
    vals = []
    sels = []
    for s_grp in range(_TOPK):
        x = ml_ref[:, s_grp * _N_SKILLS:(s_grp + 1) * _N_SKILLS]
        m = jnp.max(x, axis=1, keepdims=True)
        e = jnp.exp(x - m)
        p = e / jnp.sum(e, axis=1, keepdims=True)
        w = p
        v_k = []
        s_k = []
        for _ in range(_TOPK):
            v = jnp.max(w, axis=1, keepdims=True)
            cand = jnp.where(w == v, iota, _N_SKILLS)
            i = jnp.min(cand, axis=1, keepdims=True)
            sel = iota == i
            v_k.append(v)
            s_k.append(sel)
            w = jnp.where(sel, -jnp.inf, w)
        vals.append(v_k)
        sels.append(s_k)
    outs = []
    for s in range(_TOPK):
        acc = jnp.zeros((n, _N_SKILLS), jnp.float32)
        for k in range(_TOPK):
            denom = vals[0][k] + vals[1][k] + vals[2][k]
            acc = acc + jnp.where(sels[s][k], vals[s][k] / denom, 0.0)
        outs.append(acc)
    outs.append(jnp.zeros((n, _N_SKILLS), jnp.float32))
    out_ref[...] = jnp.concatenate(outs, axis=1)


def _build_table(module_logits):
    return pl.pallas_call(
        _table_body,
        out_shape=jax.ShapeDtypeStruct((_N_TASKS, 4 * _N_SKILLS), jnp.float32),
    )(module_logits)


_D = 4 * _N_SKILLS
_NW = 32
_CHUNK = 128


def _make_gather_body(batch):
    b_per_w = batch // _NW
    n_chunks = max(1, b_per_w // _CHUNK)
    chunk = b_per_w // n_chunks

    def body(table_hbm, idx_hbm, out_hbm, idx_v, buf0, buf1, g0, g1, w0, w1):
        wid = lax.axis_index("s") * 2 + lax.axis_index("c")
        base = wid * b_per_w
        pltpu.sync_copy(idx_hbm.at[pl.ds(base, b_per_w)], idx_v)
        bufs = (buf0, buf1)
        gsems = (g0, g1)
        wsems = (w0, w1)

        def start_gather(c):
            return pltpu.async_copy(
                table_hbm.at[idx_v.at[pl.ds(c * chunk, chunk)]],
                bufs[c % 2],
                gsems[c % 2],
            )

        def start_write(c):
            return pltpu.async_copy(
                bufs[c % 2],
                out_hbm.at[pl.ds(base + c * chunk, chunk)],
                wsems[c % 2],
            )

        gathers = [None] * n_chunks
        writes = [None] * n_chunks
        gathers[0] = start_gather(0)
        for c in range(n_chunks):
            if c + 1 < n_chunks:
                if c >= 1:
                    writes[c - 1].wait()
                gathers[c + 1] = start_gather(c + 1)
            gathers[c].wait()
            writes[c] = start_write(c)
        for w in writes[-2:]:
            w.wait()

    return body, chunk, b_per_w


def _gather(table, task_ids):
    batch = task_ids.shape[0]
    body, chunk, b_per_w = _make_gather_body(batch)
    mesh = plsc.VectorSubcoreMesh(core_axis_name="c", subcore_axis_name="s")
    grab = pl.kernel(
        body,
        out_type=jax.ShapeDtypeStruct((batch, _D), jnp.float32),
        mesh=mesh,
        scratch_types=[
            pltpu.VMEM((b_per_w,), jnp.int32),
            pltpu.VMEM((chunk, _D), jnp.float32),
            pltpu.VMEM((chunk, _D), jnp.float32),
            pltpu.SemaphoreType.DMA,
            pltpu.SemaphoreType.DMA,
            pltpu.SemaphoreType.DMA,
            pltpu.SemaphoreType.DMA,
        ],
    )
    return grab(table, task_ids)


def _expand(flat):
    batch = flat.shape[0]
    top = flat.reshape(batch, 4, _N_SKILLS)
    tail = jnp.zeros((batch, _N_SPLITS - 4, _N_SKILLS), jnp.float32)
    return jnp.concatenate([top, tail], axis=1)


def kernel(task_ids, module_logits):
    table = _build_table(module_logits)
    n_pieces = 2
    piece = _BATCH // n_pieces
    outs = []
    for p in range(n_pieces):
        flat = _gather(table, task_ids[p * piece:(p + 1) * piece])
        outs.append(_expand(flat))
    return jnp.concatenate(outs, axis=0)
